# Initial kernel scaffold; baseline (speedup 1.0000x reference)
import jax
import jax.numpy as jnp
from jax.experimental import pallas as pl


def kernel(input, table, W_a, b_a, W_b, b_b):
    raise NotImplementedError("write your pallas kernel here")


# speedup vs baseline: 10.4619x; 10.4619x over previous
import jax
import jax.numpy as jnp
from jax.experimental import pallas as pl
from jax.experimental.pallas import tpu as pltpu

_B = 16384
_L = 40
_V = 5
_E = 5
_FC = 200
_BB = 2048


def _body(inp_ref, table_ref, wa_ref, ba_ref, wb_ref, bb_ref, out_ref, q_ref):
    @pl.when(pl.program_id(0) == 0)
    def _():
        ri = jax.lax.broadcasted_iota(jnp.int32, (_FC, _FC), 0)
        ci = jax.lax.broadcasted_iota(jnp.int32, (_FC, _FC), 1)
        blk = (ci // _E) == (ri // _E)
        s = jnp.zeros((_FC, _FC), jnp.float32)
        for v in range(_V):
            rv = (ri % _V) == v
            for e in range(_E):
                m = blk & rv & ((ci % _E) == e)
                s = jnp.where(m, table_ref[v, e], s)
        w = wa_ref[...] + wb_ref[...]
        q_ref[...] = jax.lax.dot_general(
            s, w, (((1,), (1,)), ((), ())), preferred_element_type=jnp.float32
        )

    rep = jnp.repeat(inp_ref[...], _E, axis=1)
    ci2 = jax.lax.broadcasted_iota(jnp.int32, (_BB, _FC), 1)
    m = (rep == ci2 % _V).astype(jnp.float32)
    bias = ba_ref[0, :] + bb_ref[0, :]
    out_ref[...] = (
        jax.lax.dot_general(
            m, q_ref[...], (((1,), (0,)), ((), ())), preferred_element_type=jnp.float32
        )
        + bias[None, :]
    )


def kernel(input, table, W_a, b_a, W_b, b_b):
    grid = _B // _BB
    return pl.pallas_call(
        _body,
        grid=(grid,),
        in_specs=[
            pl.BlockSpec((_BB, _L), lambda i: (i, 0)),
            pl.BlockSpec(memory_space=pltpu.SMEM),
            pl.BlockSpec((_FC, _FC), lambda i: (0, 0)),
            pl.BlockSpec((1, _FC), lambda i: (0, 0)),
            pl.BlockSpec((_FC, _FC), lambda i: (0, 0)),
            pl.BlockSpec((1, _FC), lambda i: (0, 0)),
        ],
        out_specs=pl.BlockSpec((_BB, _FC), lambda i: (i, 0)),
        out_shape=jax.ShapeDtypeStruct((_B, _FC), jnp.float32),
        scratch_shapes=[pltpu.VMEM((_FC, _FC), jnp.float32)],
    )(input.astype(jnp.int32), table, W_a, b_a.reshape(1, _FC), W_b, b_b.reshape(1, _FC))


# speedup vs baseline: 63.8177x; 6.1000x over previous
import jax
import jax.numpy as jnp
from jax.experimental import pallas as pl
from jax.experimental.pallas import tpu as pltpu

_B = 16384
_L = 40
_V = 5
_E = 5
_FC = 200
_BB = 2048


def _body(inp_ref, table_ref, wa_ref, ba_ref, wb_ref, bb_ref, out_ref, q_ref, r_ref):
    @pl.when(pl.program_id(0) == 0)
    def _():
        ri = jax.lax.broadcasted_iota(jnp.int32, (_FC, _FC), 0)
        ci = jax.lax.broadcasted_iota(jnp.int32, (_FC, _FC), 1)
        blk = (ci // _E) == (ri // _E)
        s = jnp.zeros((_FC, _FC), jnp.float32)
        for v in range(_V):
            rv = (ri % _V) == v
            for e in range(_E):
                m = blk & rv & ((ci % _E) == e)
                s = jnp.where(m, table_ref[v, e], s)
        w = wa_ref[...] + wb_ref[...]
        q_ref[...] = jax.lax.dot_general(
            s, w, (((1,), (1,)), ((), ())), preferred_element_type=jnp.float32
        ).astype(jnp.bfloat16)
        rl = jax.lax.broadcasted_iota(jnp.int32, (_L, _FC), 0)
        rc = jax.lax.broadcasted_iota(jnp.int32, (_L, _FC), 1)
        r_ref[...] = (rc // _E == rl).astype(jnp.bfloat16)

    rep = jax.lax.dot_general(
        inp_ref[...].astype(jnp.bfloat16),
        r_ref[...],
        (((1,), (0,)), ((), ())),
        preferred_element_type=jnp.float32,
    )
    ci2 = jax.lax.broadcasted_iota(jnp.int32, (_BB, _FC), 1)
    m = (rep == (ci2 % _V).astype(jnp.float32)).astype(jnp.bfloat16)
    bias = ba_ref[0, :] + bb_ref[0, :]
    out_ref[...] = (
        jax.lax.dot_general(
            m, q_ref[...], (((1,), (0,)), ((), ())), preferred_element_type=jnp.float32
        )
        + bias[None, :]
    )


def kernel(input, table, W_a, b_a, W_b, b_b):
    grid = _B // _BB
    return pl.pallas_call(
        _body,
        grid=(grid,),
        in_specs=[
            pl.BlockSpec((_BB, _L), lambda i: (i, 0)),
            pl.BlockSpec(memory_space=pltpu.SMEM),
            pl.BlockSpec((_FC, _FC), lambda i: (0, 0)),
            pl.BlockSpec((1, _FC), lambda i: (0, 0)),
            pl.BlockSpec((_FC, _FC), lambda i: (0, 0)),
            pl.BlockSpec((1, _FC), lambda i: (0, 0)),
        ],
        out_specs=pl.BlockSpec((_BB, _FC), lambda i: (i, 0)),
        out_shape=jax.ShapeDtypeStruct((_B, _FC), jnp.float32),
        scratch_shapes=[
            pltpu.VMEM((_FC, _FC), jnp.bfloat16),
            pltpu.VMEM((_L, _FC), jnp.bfloat16),
        ],
    )(input.astype(jnp.int32), table, W_a, b_a.reshape(1, _FC), W_b, b_b.reshape(1, _FC))
